# Initial kernel scaffold; baseline (speedup 1.0000x reference)
import jax
import jax.numpy as jnp
from jax.experimental import pallas as pl


def kernel(ngrams_2, ngrams_3, ngrams_4, tables_2, tables_3, tables_4):
    raise NotImplementedError("write your pallas kernel here")


# speedup vs baseline: 1.8339x; 1.8339x over previous
import jax
import jax.numpy as jnp
from jax import lax
from jax.experimental import pallas as pl
from jax.experimental.pallas import tpu as pltpu
from jax.experimental.pallas import tpu_sc as plsc

_MIN_N, _MAX_N = 2, 4
_NUM_HEADS = 4
_TABLE_SIZE = 100000
_EMBED_DIM = 64
_B, _S = 1024, 50
_BS = _B * _S


def _get_prime(n):
    def is_prime(x):
        if x < 2:
            return False
        for i in range(2, int(x ** 0.5) + 1):
            if x % i == 0:
                return False
        return True
    while not is_prime(n):
        n += 1
    return n


_BASES = [_get_prime(i * 100 + 31) for i in range(_NUM_HEADS)]
_MODULI = [_get_prime(_TABLE_SIZE + i * 1000) for i in range(_NUM_HEADS)]

_C0 = {n: [[pow(_BASES[k], i, _MODULI[k]) for i in range(n)]
           for k in range(_NUM_HEADS)] for n in range(_MIN_N, _MAX_N + 1)}
_C1 = {n: [[(256 * pow(_BASES[k], i, _MODULI[k])) % _MODULI[k] for i in range(n)]
           for k in range(_NUM_HEADS)] for n in range(_MIN_N, _MAX_N + 1)}
_ROW0 = {2: 0, 3: 2, 4: 5}

_NC, _NS = 2, 16
_NW = _NC * _NS
_C = _BS // _NW
_CH = 80
_NCH = _C // _CH
_NJ = (_MAX_N - _MIN_N + 1) * _NUM_HEADS


def _body(ng_hbm, t2_hbm, t3_hbm, t4_hbm, out_hbm, ng_v, idx_v, buf_v, gsem):
    i32 = jnp.int32
    w = lax.axis_index("s") * i32(_NC) + lax.axis_index("c")
    base_row = w * i32(_C)

    pltpu.sync_copy(ng_hbm.at[w], ng_v)

    def hstep(c, carry):
        for t5 in range(_CH // 16):
            off = c * i32(_CH) + i32(t5 * 16)
            for n in range(_MIN_N, _MAX_N + 1):
                r0 = _ROW0[n]
                gs = [ng_v[r0 + i, pl.ds(off, 16)] for i in range(n)]
                g1 = [g >> 8 for g in gs]
                g0 = [g & 255 for g in gs]
                for k in range(_NUM_HEADS):
                    j = (n - _MIN_N) * _NUM_HEADS + k
                    acc = g1[0] * _C1[n][k][0] + g0[0] * _C0[n][k][0]
                    for i in range(1, n):
                        acc = acc + g1[i] * _C1[n][k][i] + g0[i] * _C0[n][k][i]
                    h = acc % _MODULI[k]
                    h = jnp.where(h >= _TABLE_SIZE, h - _TABLE_SIZE, h)
                    idx_v[j, c, pl.ds(t5 * 16, 16)] = h + k * _TABLE_SIZE
        return carry

    lax.fori_loop(jnp.int32(0), jnp.int32(_NCH), hstep, jnp.int32(0))

    tabs = (t2_hbm, t3_hbm, t4_hbm)
    for j in range(_NJ):
        tab = tabs[j // _NUM_HEADS]

        def gstep(c, carry, j=j, tab=tab):
            pltpu.async_copy(tab.at[idx_v.at[jnp.int32(j), c]], buf_v, gsem).wait()
            pltpu.sync_copy(
                buf_v,
                out_hbm.at[pl.ds(base_row + c * jnp.int32(_CH), _CH), jnp.int32(j)])
            return carry

        lax.fori_loop(jnp.int32(0), jnp.int32(_NCH), gstep, jnp.int32(0))


def kernel(ngrams_2, ngrams_3, ngrams_4, tables_2, tables_3, tables_4):
    ng2 = ngrams_2.reshape(_BS, 2).astype(jnp.int32)
    ng3 = ngrams_3.reshape(_BS, 3).astype(jnp.int32)
    ng4 = ngrams_4.reshape(_BS, 4).astype(jnp.int32)
    ngall = jnp.concatenate([ng2, ng3, ng4], axis=1)
    ngt = ngall.reshape(_NW, _C, 9).transpose(0, 2, 1)

    t2 = tables_2.reshape(_NUM_HEADS * _TABLE_SIZE, _EMBED_DIM)
    t3 = tables_3.reshape(_NUM_HEADS * _TABLE_SIZE, _EMBED_DIM)
    t4 = tables_4.reshape(_NUM_HEADS * _TABLE_SIZE, _EMBED_DIM)

    mesh = plsc.VectorSubcoreMesh(core_axis_name="c", subcore_axis_name="s",
                                  num_cores=_NC, num_subcores=_NS)
    run = pl.kernel(
        _body,
        out_type=jax.ShapeDtypeStruct((_BS, _NJ, _EMBED_DIM), jnp.float32),
        mesh=mesh,
        scratch_types=[
            pltpu.VMEM((9, _C), jnp.int32),
            pltpu.VMEM((_NJ, _NCH, _CH), jnp.int32),
            pltpu.VMEM((_CH, _EMBED_DIM), jnp.float32),
            pltpu.SemaphoreType.DMA,
        ],
        compiler_params=pltpu.CompilerParams(use_tc_tiling_on_sc=False),
    )
    out = run(ngt, t2, t3, t4)
    return out.reshape(_B, _S, _NJ * _EMBED_DIM)


# speedup vs baseline: 2.1709x; 1.1838x over previous
import jax
import jax.numpy as jnp
from jax import lax
from jax.experimental import pallas as pl
from jax.experimental.pallas import tpu as pltpu
from jax.experimental.pallas import tpu_sc as plsc

_MIN_N, _MAX_N = 2, 4
_NUM_HEADS = 4
_TABLE_SIZE = 100000
_EMBED_DIM = 64
_B, _S = 1024, 50
_BS = _B * _S


def _get_prime(n):
    def is_prime(x):
        if x < 2:
            return False
        for i in range(2, int(x ** 0.5) + 1):
            if x % i == 0:
                return False
        return True
    while not is_prime(n):
        n += 1
    return n


_BASES = [_get_prime(i * 100 + 31) for i in range(_NUM_HEADS)]
_MODULI = [_get_prime(_TABLE_SIZE + i * 1000) for i in range(_NUM_HEADS)]

_C0 = {n: [[pow(_BASES[k], i, _MODULI[k]) for i in range(n)]
           for k in range(_NUM_HEADS)] for n in range(_MIN_N, _MAX_N + 1)}
_C1 = {n: [[(256 * pow(_BASES[k], i, _MODULI[k])) % _MODULI[k] for i in range(n)]
           for k in range(_NUM_HEADS)] for n in range(_MIN_N, _MAX_N + 1)}
_ROW0 = {2: 0, 3: 2, 4: 5}

_NC, _NS = 2, 16
_NW = _NC * _NS
_C = _BS // _NW
_CH = 80
_NCH = _C // _CH
_NJ = (_MAX_N - _MIN_N + 1) * _NUM_HEADS


def _body(ng_hbm, t2_hbm, t3_hbm, t4_hbm, out_hbm, ng_v, idx_v, buf0_v, buf1_v,
          gsem0, gsem1, wsem0, wsem1):
    i32 = jnp.int32
    w = lax.axis_index("s") * i32(_NC) + lax.axis_index("c")
    base_row = w * i32(_C)
    bufs = (buf0_v, buf1_v)
    gsems = (gsem0, gsem1)
    wsems = (wsem0, wsem1)

    pltpu.sync_copy(ng_hbm.at[w], ng_v)

    def hstep(c, carry):
        for t5 in range(_CH // 16):
            off = c * i32(_CH) + i32(t5 * 16)
            for n in range(_MIN_N, _MAX_N + 1):
                r0 = _ROW0[n]
                gs = [ng_v[r0 + i, pl.ds(off, 16)] for i in range(n)]
                g1 = [g >> 8 for g in gs]
                g0 = [g & 255 for g in gs]
                for k in range(_NUM_HEADS):
                    j = (n - _MIN_N) * _NUM_HEADS + k
                    acc = g1[0] * _C1[n][k][0] + g0[0] * _C0[n][k][0]
                    for i in range(1, n):
                        acc = acc + g1[i] * _C1[n][k][i] + g0[i] * _C0[n][k][i]
                    h = acc % _MODULI[k]
                    h = jnp.where(h >= _TABLE_SIZE, h - _TABLE_SIZE, h)
                    idx_v[j, c, pl.ds(t5 * 16, 16)] = h + k * _TABLE_SIZE
        return carry

    lax.fori_loop(jnp.int32(0), jnp.int32(_NCH), hstep, jnp.int32(0))

    tabs = (t2_hbm, t3_hbm, t4_hbm)

    def fire_gather(j, c, b):
        tab = tabs[j // _NUM_HEADS]
        pltpu.async_copy(tab.at[idx_v.at[jnp.int32(j), c]], bufs[b], gsems[b])

    def fire_write(j, c, b):
        pltpu.make_async_copy(
            bufs[b],
            out_hbm.at[pl.ds(base_row + c * jnp.int32(_CH), _CH), jnp.int32(j)],
            wsems[b]).start()

    def drain_write(j, c, b):
        pltpu.make_async_copy(
            bufs[b],
            out_hbm.at[pl.ds(base_row + c * jnp.int32(_CH), _CH), jnp.int32(j)],
            wsems[b]).wait()

    def drain_gather(j, c, b):
        tab = tabs[j // _NUM_HEADS]
        pltpu.make_async_copy(tab.at[idx_v.at[jnp.int32(j), c]], bufs[b],
                              gsems[b]).wait()

    fire_gather(0, jnp.int32(0), 0)
    fire_gather(0, jnp.int32(1), 1)
    for j in range(_NJ):

        def pstep(c2, carry, j=j):
            for b in range(2):
                c = c2 * i32(2) + i32(b)
                drain_gather(j, c, b)
                fire_write(j, c, b)
            for b in range(2):
                c = c2 * i32(2) + i32(b)
                drain_write(j, c, b)

                @pl.when(c2 < i32(_NCH // 2 - 1))
                def _(j=j, c=c, b=b):
                    fire_gather(j, c + i32(2), b)
            return carry

        lax.fori_loop(jnp.int32(0), jnp.int32(_NCH // 2), pstep, jnp.int32(0))
        if j + 1 < _NJ:
            fire_gather(j + 1, jnp.int32(0), 0)
            fire_gather(j + 1, jnp.int32(1), 1)


def kernel(ngrams_2, ngrams_3, ngrams_4, tables_2, tables_3, tables_4):
    ng2 = ngrams_2.reshape(_BS, 2).astype(jnp.int32)
    ng3 = ngrams_3.reshape(_BS, 3).astype(jnp.int32)
    ng4 = ngrams_4.reshape(_BS, 4).astype(jnp.int32)
    ngall = jnp.concatenate([ng2, ng3, ng4], axis=1)
    ngq = ngall.reshape(_B, _S, 9).transpose(1, 0, 2).reshape(_BS, 9)
    ngt = ngq.reshape(_NW, _C, 9).transpose(0, 2, 1)

    t2 = tables_2.reshape(_NUM_HEADS * _TABLE_SIZE, _EMBED_DIM)
    t3 = tables_3.reshape(_NUM_HEADS * _TABLE_SIZE, _EMBED_DIM)
    t4 = tables_4.reshape(_NUM_HEADS * _TABLE_SIZE, _EMBED_DIM)

    mesh = plsc.VectorSubcoreMesh(core_axis_name="c", subcore_axis_name="s",
                                  num_cores=_NC, num_subcores=_NS)
    run = pl.kernel(
        _body,
        out_type=jax.ShapeDtypeStruct((_BS, _NJ, _EMBED_DIM), jnp.float32),
        mesh=mesh,
        scratch_types=[
            pltpu.VMEM((9, _C), jnp.int32),
            pltpu.VMEM((_NJ, _NCH, _CH), jnp.int32),
            pltpu.VMEM((_CH, _EMBED_DIM), jnp.float32),
            pltpu.VMEM((_CH, _EMBED_DIM), jnp.float32),
            pltpu.SemaphoreType.DMA,
            pltpu.SemaphoreType.DMA,
            pltpu.SemaphoreType.DMA,
            pltpu.SemaphoreType.DMA,
        ],
        compiler_params=pltpu.CompilerParams(use_tc_tiling_on_sc=False),
    )
    out = run(ngt, t2, t3, t4)
    out = out.reshape(_S, _B, _NJ * _EMBED_DIM)
    return out.transpose(1, 0, 2)
